# baseline (speedup 1.0000x reference)
import functools

import jax
import jax.numpy as jnp
from jax import lax
from jax.experimental import pallas as pl
from jax.experimental.pallas import tpu as pltpu
from jax.experimental.pallas import tpu_sc as plsc

N, P3, W3, CKV = 8, 49, 64, 384
TOPK = 4
D = W3 * CKV
ROWS = N * P3
B = N * P3 * TOPK
NC, NS = 2, 16
NW = NC * NS
RPW = B // NW

_mesh = plsc.VectorSubcoreMesh(core_axis_name="c", subcore_axis_name="s")


@functools.partial(
    pl.kernel,
    mesh=_mesh,
    out_type=jax.ShapeDtypeStruct((B, D), jnp.float32),
    scratch_types=[
        pltpu.VMEM((RPW, 1), jnp.int32),
        pltpu.VMEM((1, D), jnp.float32),
        pltpu.SemaphoreType.DMA,
    ],
)
def _sc_gather(gidx_hbm, table_hbm, out_hbm, idx_v, buf, sem):
    wid = lax.axis_index("s") * NC + lax.axis_index("c")
    base = wid * RPW
    pltpu.sync_copy(gidx_hbm.at[wid], idx_v)

    def body(i, carry):
        pltpu.async_copy(table_hbm.at[idx_v.at[i]], buf, sem).wait()
        pltpu.sync_copy(buf, out_hbm.at[pl.ds(base + i, 1)])
        return carry

    lax.fori_loop(0, RPW, body, 0)


def kernel(r_idx, qkv):
    gidx = (
        jnp.arange(N, dtype=jnp.int32)[:, None, None] * P3
        + r_idx.astype(jnp.int32)
    ).reshape(NW, RPW, 1)
    table = qkv.reshape(ROWS, D)
    out = _sc_gather(gidx, table)
    return out.reshape(N, P3, TOPK, W3, CKV)


# speedup vs baseline: 1.1176x; 1.1176x over previous
import functools

import jax
import jax.numpy as jnp
from jax import lax
from jax.experimental import pallas as pl
from jax.experimental.pallas import tpu as pltpu
from jax.experimental.pallas import tpu_sc as plsc

N, P3, W3, CKV = 8, 49, 64, 384
TOPK = 4
D = W3 * CKV
ROWS = N * P3
B = N * P3 * TOPK
NC, NS = 2, 16
NW = NC * NS
RPW = B // NW

_mesh = plsc.VectorSubcoreMesh(core_axis_name="c", subcore_axis_name="s")


NBUF = 4


@functools.partial(
    pl.kernel,
    mesh=_mesh,
    out_type=jax.ShapeDtypeStruct((B, D), jnp.float32),
    scratch_types=[
        pltpu.VMEM((RPW, 1), jnp.int32),
    ]
    + [pltpu.VMEM((1, D), jnp.float32) for _ in range(NBUF)]
    + [pltpu.SemaphoreType.DMA for _ in range(2 * NBUF)],
)
def _sc_gather(gidx_hbm, table_hbm, out_hbm, idx_v, *scr):
    bufs = scr[:NBUF]
    gsems = scr[NBUF : 2 * NBUF]
    wsems = scr[2 * NBUF :]
    wid = lax.axis_index("s") * NC + lax.axis_index("c")
    base = wid * RPW
    pltpu.sync_copy(gidx_hbm.at[wid], idx_v)

    def fire_gather(i, slot):
        pltpu.async_copy(table_hbm.at[idx_v.at[i]], bufs[slot], gsems[slot])

    def wait_gather(i, slot):
        pltpu.make_async_copy(
            table_hbm.at[idx_v.at[i]], bufs[slot], gsems[slot]
        ).wait()

    def fire_write(i, slot):
        pltpu.async_copy(bufs[slot], out_hbm.at[pl.ds(base + i, 1)], wsems[slot])

    def wait_write(i, slot):
        pltpu.make_async_copy(
            bufs[slot], out_hbm.at[pl.ds(base + i, 1)], wsems[slot]
        ).wait()

    fire_gather(0, 0)
    fire_gather(1, 1)
    fire_gather(2, 2); wait_gather(0, 0); fire_write(0, 0)
    fire_gather(3, 3); wait_gather(1, 1); fire_write(1, 1)
    wait_write(0, 0); fire_gather(4, 0); wait_gather(2, 2); fire_write(2, 2)
    wait_write(1, 1); fire_gather(5, 1); wait_gather(3, 3); fire_write(3, 3)

    def body(g, carry):
        for k in range(4):
            i = 4 * g + k
            fslot = (k + 2) % 4
            wait_write(i - 2, fslot)
            fire_gather(i + 2, fslot)
            wait_gather(i, k)
            fire_write(i, k)
        return carry

    lax.fori_loop(1, 11, body, 0)
    wait_write(42, 2); fire_gather(46, 2); wait_gather(44, 0); fire_write(44, 0)
    wait_write(43, 3); fire_gather(47, 3); wait_gather(45, 1); fire_write(45, 1)
    wait_write(44, 0); fire_gather(48, 0); wait_gather(46, 2); fire_write(46, 2)
    wait_gather(47, 3); fire_write(47, 3)
    wait_gather(48, 0); fire_write(48, 0)
    wait_write(45, 1)
    wait_write(46, 2)
    wait_write(47, 3)
    wait_write(48, 0)


def kernel(r_idx, qkv):
    gidx = (
        jnp.arange(N, dtype=jnp.int32)[:, None, None] * P3
        + r_idx.astype(jnp.int32)
    ).reshape(NW, RPW, 1)
    table = qkv.reshape(ROWS, D)
    out = _sc_gather(gidx, table)
    return out.reshape(N, P3, TOPK, W3, CKV)
